# baseline (speedup 1.0000x reference)
import functools

import jax
import jax.numpy as jnp
from jax import lax
from jax.experimental import pallas as pl
from jax.experimental.pallas import tpu as pltpu
from jax.experimental.pallas import tpu_sc as plsc

D = 64
NW = 32
CH = 512
RW = 512


def _retile_body(wt_ref, std_ref):
    xt = wt_ref[...].T
    std_ref[:, 0:D] = xt
    std_ref[:, D:2 * D] = xt


@jax.jit
def _embed(ids_flat, w_t):
    n = ids_flat.shape[0]
    vocab = w_t.shape[1]
    n_blk = (vocab + RW - 1) // RW
    per_w = n // NW
    n_ch = per_w // CH

    std = pl.pallas_call(
        _retile_body,
        grid=(n_blk,),
        in_specs=[pl.BlockSpec((D, RW), lambda c: (0, c))],
        out_specs=pl.BlockSpec((RW, 2 * D), lambda c: (c, 0)),
        out_shape=jax.ShapeDtypeStruct((n_blk * RW, 2 * D), jnp.float32),
    )(w_t)

    mesh = plsc.VectorSubcoreMesh(core_axis_name="c", subcore_axis_name="s")

    @functools.partial(
        pl.kernel,
        mesh=mesh,
        out_type=jax.ShapeDtypeStruct((n, 2 * D), jnp.float32),
        scratch_types=[
            pltpu.VMEM((CH,), jnp.int32),
            pltpu.VMEM((CH, 2 * D), jnp.float32),
            pltpu.SemaphoreType.DMA,
        ],
    )
    def gather(std_hbm, ids_hbm, out_hbm, idx_v, rows_v, sem):
        wid = lax.axis_index("s") * 2 + lax.axis_index("c")
        base = wid * per_w

        def body(k, carry):
            row0 = pl.multiple_of(base + k * CH, 8)
            pltpu.sync_copy(ids_hbm.at[pl.ds(row0, CH)], idx_v)
            pltpu.async_copy(std_hbm.at[idx_v], rows_v, sem).wait()
            pltpu.sync_copy(rows_v, out_hbm.at[pl.ds(row0, CH)])
            return carry

        lax.fori_loop(0, n_ch, body, 0)

    return gather(std, ids_flat)


def kernel(input_ids, weight):
    ids_flat = input_ids.reshape(-1).astype(jnp.int32)
    out = _embed(ids_flat, weight.T)
    return out[:, :D].reshape(input_ids.shape + (D,))
